# baseline (speedup 1.0000x reference)
import functools

import jax
import jax.numpy as jnp
from jax import lax
from jax.experimental import pallas as pl
from jax.experimental.pallas import tpu as pltpu
from jax.experimental.pallas import tpu_sc as plsc

ALPHA = 0.1
B = 16384
D = 16
NC = 2
NS = 16
NW = NC * NS
BPW = B // NW
K = 128
NCHUNK = BPW // K
UNROLL = 8

_mesh = plsc.VectorSubcoreMesh(core_axis_name="c", subcore_axis_name="s")


@functools.partial(
    pl.kernel,
    mesh=_mesh,
    out_type=jax.ShapeDtypeStruct((B, D), jnp.float32),
    scratch_types=[
        pltpu.VMEM((NCHUNK, K), jnp.int32),
        pltpu.VMEM((BPW, D), jnp.float32),
        pltpu.VMEM((BPW, D), jnp.float32),
        pltpu.VMEM((BPW, D), jnp.float32),
        pltpu.SemaphoreType.DMA,
    ],
    compiler_params=pltpu.CompilerParams(use_tc_tiling_on_sc=False),
)
def _noise_kernel(x_hbm, idx_hbm, std_hbm, eps_hbm, out_hbm,
                  idx_v, rows_v, x_v, eps_v, sem):
    wid = lax.axis_index("s") * NC + lax.axis_index("c")
    base = wid * BPW

    pltpu.sync_copy(idx_hbm.at[pl.ds(wid * NCHUNK, NCHUNK)], idx_v)

    copies = []
    for j in range(NCHUNK):
        copies.append(
            pltpu.async_copy(std_hbm.at[idx_v.at[j]],
                             rows_v.at[pl.ds(j * K, K)], sem))

    pltpu.sync_copy(x_hbm.at[pl.ds(base, BPW)], x_v)
    pltpu.sync_copy(eps_hbm.at[pl.ds(base, BPW)], eps_v)

    for c in copies:
        c.wait()

    def body(i, carry):
        for r in range(UNROLL):
            row = i * UNROLL + r
            eps_v[row] = x_v[row] + (eps_v[row] * rows_v[row]) * ALPHA
        return carry

    lax.fori_loop(0, BPW // UNROLL, body, 0)

    pltpu.sync_copy(eps_v, out_hbm.at[pl.ds(base, BPW)])


def kernel(x, newY, std, eps):
    idx2 = newY.reshape(NW * NCHUNK, K)
    return _noise_kernel(x, idx2, std, eps)


# speedup vs baseline: 1.0005x; 1.0005x over previous
import functools

import jax
import jax.numpy as jnp
from jax import lax
from jax.experimental import pallas as pl
from jax.experimental.pallas import tpu as pltpu
from jax.experimental.pallas import tpu_sc as plsc

ALPHA = 0.1
B = 16384
D = 16
NC = 2
NS = 16
NW = NC * NS
BPW = B // NW
K = 128
NCHUNK = BPW // K
LCH = 16

_mesh = plsc.VectorSubcoreMesh(core_axis_name="c", subcore_axis_name="s")


@functools.partial(
    pl.kernel,
    mesh=_mesh,
    out_type=jax.ShapeDtypeStruct((B, D), jnp.float32),
    scratch_types=[
        pltpu.VMEM((NCHUNK, K), jnp.int32),
        pltpu.VMEM((BPW, D), jnp.float32),
        pltpu.SemaphoreType.DMA,
    ],
    compiler_params=pltpu.CompilerParams(use_tc_tiling_on_sc=False),
)
def _gather_kernel(idx_hbm, std_hbm, g_hbm, idx_v, rows_v, sem):
    wid = lax.axis_index("s") * NC + lax.axis_index("c")
    base = wid * BPW

    pltpu.sync_copy(idx_hbm.at[pl.ds(wid * NCHUNK, NCHUNK)], idx_v)

    copies = []
    for j in range(NCHUNK):
        copies.append(
            pltpu.async_copy(std_hbm.at[idx_v.at[j]],
                             rows_v.at[pl.ds(j * K, K)], sem))
    for c in copies:
        c.wait()

    pltpu.sync_copy(rows_v, g_hbm.at[pl.ds(base, BPW)])


@functools.partial(
    pl.kernel,
    mesh=_mesh,
    out_type=jax.ShapeDtypeStruct((D, B), jnp.float32),
    scratch_types=[
        pltpu.VMEM((D, BPW), jnp.float32),
        pltpu.VMEM((D, BPW), jnp.float32),
        pltpu.VMEM((D, BPW), jnp.float32),
    ],
    compiler_params=pltpu.CompilerParams(use_tc_tiling_on_sc=True),
)
def _fma_kernel(xt_hbm, gt_hbm, epst_hbm, outt_hbm, g_v, x_v, eps_v):
    wid = lax.axis_index("s") * NC + lax.axis_index("c")
    base = wid * BPW

    pltpu.sync_copy(gt_hbm.at[:, pl.ds(base, BPW)], g_v)
    pltpu.sync_copy(xt_hbm.at[:, pl.ds(base, BPW)], x_v)
    pltpu.sync_copy(epst_hbm.at[:, pl.ds(base, BPW)], eps_v)

    def fma_chunk(c, carry):
        off = c * LCH
        for s in range(D):
            xr = x_v.at[s][pl.ds(off, LCH)]
            er = eps_v.at[s][pl.ds(off, LCH)]
            gr = g_v.at[s][pl.ds(off, LCH)]
            eps_v.at[s][pl.ds(off, LCH)] = xr + (er * gr) * ALPHA
        return carry

    lax.fori_loop(0, BPW // LCH, fma_chunk, 0)

    pltpu.sync_copy(eps_v, outt_hbm.at[:, pl.ds(base, BPW)])


def kernel(x, newY, std, eps):
    idx2 = newY.reshape(NW * NCHUNK, K)
    g = _gather_kernel(idx2, std)
    out_t = _fma_kernel(x.T, g.T, eps.T)
    return out_t.T
